# baseline (speedup 1.0000x reference)
import jax
import jax.numpy as jnp
from jax.experimental import pallas as pl
from jax.experimental.pallas import tpu as pltpu
from jax.experimental.pallas import tpu_sc as plsc

E = 8
D = 2048
DFF = 1408
DSH = 5632
S = 2048
EPS = 1e-06
BLK = 256
NB = 24
NP = NB * BLK
LANES = 128
NEG = -1e30
RT = 512
TT = 512
HT = 512
SCG = 128
DSPLIT = 4
DC = D // DSPLIT
DCW = DC // 2


def _router_body(x_ref, gwc_ref, ln_ref, misc_ref, hb_ref):
    x = x_ref[...]
    var = jnp.mean(x * x, axis=1, keepdims=True)
    h = x * jax.lax.rsqrt(var + EPS) * ln_ref[...]
    hb_ref[...] = h.astype(jnp.bfloat16)
    logits = jax.lax.dot_general(
        h, gwc_ref[...], (((1,), (1,)), ((), ())),
        preferred_element_type=jnp.float32,
        precision=jax.lax.Precision.HIGHEST)
    lane = jax.lax.broadcasted_iota(jnp.int32, logits.shape, 1)
    l1 = jnp.where(lane < E, logits, NEG)
    m1 = jnp.max(l1, axis=1, keepdims=True)
    a1 = jnp.min(jnp.where(l1 == m1, lane, LANES), axis=1, keepdims=True)
    l2 = jnp.where(lane == a1, NEG, l1)
    m2 = jnp.max(l2, axis=1, keepdims=True)
    a2 = jnp.min(jnp.where(l2 == m2, lane, LANES), axis=1, keepdims=True)
    s2 = jnp.exp(m2 - m1)
    w0 = 1.0 / (1.0 + s2)
    w1 = 1.0 - w0
    sig = jax.nn.sigmoid(logits[:, E:E + 1])
    misc_ref[...] = (jnp.where(lane == 0, w0, 0.0)
                     + jnp.where(lane == 1, w1, 0.0)
                     + jnp.where(lane == 2, a1.astype(jnp.float32), 0.0)
                     + jnp.where(lane == 3, a2.astype(jnp.float32), 0.0)
                     + jnp.where(lane == 4, sig, 0.0))


def _router(x, gwc, ln2):
    return pl.pallas_call(
        _router_body,
        grid=(S // TT,),
        in_specs=[
            pl.BlockSpec((TT, D), lambda i: (i, 0)),
            pl.BlockSpec((LANES, D), lambda i: (0, 0)),
            pl.BlockSpec((1, D), lambda i: (0, 0)),
        ],
        out_specs=[
            pl.BlockSpec((TT, LANES), lambda i: (i, 0)),
            pl.BlockSpec((TT, D), lambda i: (i, 0)),
        ],
        out_shape=[
            jax.ShapeDtypeStruct((S, LANES), jnp.float32),
            jax.ShapeDtypeStruct((S, D), jnp.bfloat16),
        ],
    )(x, gwc, ln2)


def _onehots(m):
    lane = jax.lax.broadcasted_iota(jnp.int32, (S, LANES), 1)
    e0 = m[:, 2:3].astype(jnp.int32)
    e1 = m[:, 3:4].astype(jnp.int32)
    return lane == e0, lane == e1


def _cumsum_body(misc_ref, c_ref):
    i = pl.program_id(0)
    o0, o1 = _onehots(misc_ref[...])
    onehot = jnp.concatenate(
        [o0.astype(jnp.bfloat16), o1.astype(jnp.bfloat16)], axis=0)
    row = jax.lax.broadcasted_iota(jnp.int32, (RT, 2 * S), 0) + i * RT
    col = jax.lax.broadcasted_iota(jnp.int32, (RT, 2 * S), 1)
    tri = (col <= row).astype(jnp.bfloat16)
    c_ref[...] = jax.lax.dot_general(
        tri, onehot, (((1,), (0,)), ((), ())),
        preferred_element_type=jnp.float32)


def _cumsum(misc):
    return pl.pallas_call(
        _cumsum_body,
        grid=(2 * S // RT,),
        in_specs=[pl.BlockSpec((S, LANES), lambda i: (0, 0))],
        out_specs=pl.BlockSpec((RT, LANES), lambda i: (i, 0)),
        out_shape=jax.ShapeDtypeStruct((2 * S, LANES), jnp.float32),
    )(misc)


def _dispatch_body(misc_ref, c_ref, pos_ref, be_ref):
    m = misc_ref[...]
    c = c_ref[...]
    o0, o1 = _onehots(m)
    counts = c[2 * S - 1:2 * S, :]
    nb = (counts.astype(jnp.int32) + BLK - 1) // BLK
    el = jax.lax.broadcasted_iota(jnp.int32, (LANES, LANES), 0)
    ec = jax.lax.broadcasted_iota(jnp.int32, (LANES, LANES), 1)
    tri = (el < ec).astype(jnp.float32)
    cumnb = jax.lax.dot_general(
        nb.astype(jnp.float32), tri, (((1,), (0,)), ((), ())),
        preferred_element_type=jnp.float32,
        precision=jax.lax.Precision.HIGHEST)
    base = BLK * cumnb
    p0 = jnp.sum(jnp.where(o0, c[0:S, :] - 1.0 + base, 0.0),
                 axis=1, keepdims=True)
    p1 = jnp.sum(jnp.where(o1, c[S:2 * S, :] - 1.0 + base, 0.0),
                 axis=1, keepdims=True)
    posm = jnp.concatenate([p0, p1], axis=0)
    lane2 = jax.lax.broadcasted_iota(jnp.int32, (2 * S, LANES), 1)
    lane_f = lane2.astype(jnp.float32)
    pos_ref[...] = jnp.where(lane2 < DSPLIT, DSPLIT * posm + lane_f,
                             0.0).astype(jnp.int32)
    brow = jax.lax.broadcasted_iota(jnp.int32, (32, LANES), 0)
    lane_b = jax.lax.broadcasted_iota(jnp.int32, (32, LANES), 1)
    active = jnp.where((lane_b < E) & (cumnb <= brow.astype(jnp.float32)),
                       1, 0)
    be = jnp.sum(active, axis=1, keepdims=True) - 1
    be_ref[...] = jnp.where(lane_b == 0, be, 0)


def _dispatch(misc, c):
    return pl.pallas_call(
        _dispatch_body,
        out_shape=[
            jax.ShapeDtypeStruct((2 * S, LANES), jnp.int32),
            jax.ShapeDtypeStruct((32, LANES), jnp.int32),
        ],
    )(misc, c)


def _sc_mesh():
    return plsc.VectorSubcoreMesh(core_axis_name="core",
                                  subcore_axis_name="subcore")


def _sc_scatter(hb, idx):
    n_idx = DSPLIT * 2 * S
    n_src = DSPLIT * S // SCG

    @pl.kernel(out_type=jax.ShapeDtypeStruct((NP * DSPLIT, DCW), jnp.int32),
               mesh=_sc_mesh())
    def k(x_hbm, i_hbm, o_hbm):
        def body(x_vmem, i_vmem):
            pltpu.sync_copy(x_vmem, o_hbm.at[i_vmem.at[0]])

        pltpu.emit_pipeline(
            body,
            grid=(n_idx // SCG,),
            in_specs=[
                pl.BlockSpec((SCG, DCW), lambda i: (i % n_src, 0)),
                pl.BlockSpec((1, SCG), lambda i: (0, i)),
            ],
            out_specs=[],
            core_axis_name=("core", "subcore"),
            dimension_semantics=(pltpu.PARALLEL,),
        )(x_hbm, i_hbm)

    return k(hb, idx)


def _sc_gather(ys, idx):
    n_idx = DSPLIT * 2 * S

    @pl.kernel(out_type=jax.ShapeDtypeStruct((n_idx, DCW), jnp.int32),
               mesh=_sc_mesh())
    def k(y_hbm, i_hbm, o_hbm):
        def body(i_vmem, o_vmem):
            pltpu.sync_copy(y_hbm.at[i_vmem.at[0]], o_vmem)

        pltpu.emit_pipeline(
            body,
            grid=(n_idx // SCG,),
            in_specs=[pl.BlockSpec((1, SCG), lambda i: (0, i))],
            out_specs=[pl.BlockSpec((SCG, DCW), lambda i: (i, 0))],
            core_axis_name=("core", "subcore"),
            dimension_semantics=(pltpu.PARALLEL,),
        )(i_hbm, o_hbm)

    return k(ys, idx)


def _grouped_body(be_ref, x_ref, gw_ref, uw_ref, dw_ref, y_ref):
    x = x_ref[...]
    g = jax.lax.dot_general(x, gw_ref[0], (((1,), (1,)), ((), ())),
                            preferred_element_type=jnp.float32)
    u = jax.lax.dot_general(x, uw_ref[0], (((1,), (1,)), ((), ())),
                            preferred_element_type=jnp.float32)
    p = (jax.nn.silu(g) * u).astype(jnp.bfloat16)
    y = jax.lax.dot_general(p, dw_ref[0], (((1,), (1,)), ((), ())),
                            preferred_element_type=jnp.float32)
    y_ref[...] = y.astype(jnp.bfloat16)


def _grouped_mlp(be, xs, egw, euw, edw):
    grid_spec = pltpu.PrefetchScalarGridSpec(
        num_scalar_prefetch=1,
        grid=(NB,),
        in_specs=[
            pl.BlockSpec((BLK, D), lambda b, be_ref: (b, 0)),
            pl.BlockSpec((1, DFF, D), lambda b, be_ref: (be_ref[b], 0, 0)),
            pl.BlockSpec((1, DFF, D), lambda b, be_ref: (be_ref[b], 0, 0)),
            pl.BlockSpec((1, D, DFF), lambda b, be_ref: (be_ref[b], 0, 0)),
        ],
        out_specs=pl.BlockSpec((BLK, D), lambda b, be_ref: (b, 0)),
    )
    return pl.pallas_call(
        _grouped_body,
        grid_spec=grid_spec,
        out_shape=jax.ShapeDtypeStruct((NP, D), jnp.bfloat16),
    )(be, xs, egw, euw, edw)


def _shared_body(hb_ref, sg_ref, su_ref, sd_ref, o_ref):
    @pl.when(pl.program_id(1) == 0)
    def _():
        o_ref[...] = jnp.zeros_like(o_ref)

    hb = hb_ref[...]
    g = jax.lax.dot_general(hb, sg_ref[...], (((1,), (1,)), ((), ())),
                            preferred_element_type=jnp.float32)
    u = jax.lax.dot_general(hb, su_ref[...], (((1,), (1,)), ((), ())),
                            preferred_element_type=jnp.float32)
    p = (jax.nn.silu(g) * u).astype(jnp.bfloat16)
    o_ref[...] += jax.lax.dot_general(p, sd_ref[...], (((1,), (1,)), ((), ())),
                                      preferred_element_type=jnp.float32)


def _shared(hb, sgw, suw, sdw):
    return pl.pallas_call(
        _shared_body,
        grid=(S // TT, DSH // HT),
        in_specs=[
            pl.BlockSpec((TT, D), lambda i, j: (i, 0)),
            pl.BlockSpec((HT, D), lambda i, j: (j, 0)),
            pl.BlockSpec((HT, D), lambda i, j: (j, 0)),
            pl.BlockSpec((D, HT), lambda i, j: (0, j)),
        ],
        out_specs=pl.BlockSpec((TT, D), lambda i, j: (i, 0)),
        out_shape=jax.ShapeDtypeStruct((S, D), jnp.float32),
    )(hb, sgw, suw, sdw)


def _final_body(x_ref, ysh_ref, g0_ref, g1_ref, misc_ref, o_ref):
    m = misc_ref[...]
    w0 = m[:, 0:1]
    w1 = m[:, 1:2]
    sig = m[:, 4:5]
    o_ref[...] = (x_ref[...] + sig * ysh_ref[...]
                  + w0 * g0_ref[...].astype(jnp.float32)
                  + w1 * g1_ref[...].astype(jnp.float32))


def _final(x, ysh, g0, g1, misc):
    return pl.pallas_call(
        _final_body,
        grid=(S // TT,),
        in_specs=[
            pl.BlockSpec((TT, D), lambda i: (i, 0)),
            pl.BlockSpec((TT, D), lambda i: (i, 0)),
            pl.BlockSpec((TT, D), lambda i: (i, 0)),
            pl.BlockSpec((TT, D), lambda i: (i, 0)),
            pl.BlockSpec((TT, LANES), lambda i: (i, 0)),
        ],
        out_specs=pl.BlockSpec((TT, D), lambda i: (i, 0)),
        out_shape=jax.ShapeDtypeStruct((S, D), jnp.float32),
    )(x, ysh, g0, g1, misc)


def kernel(hidden_states, ln_weight, gate_weight, expert_gate_w, expert_up_w,
           expert_down_w, shared_gate_w, shared_up_w, shared_down_w,
           shared_expert_gate_w):
    x = hidden_states.reshape(S, D)
    gwc = jnp.concatenate(
        [gate_weight, shared_expert_gate_w,
         jnp.zeros((LANES - E - 1, D), jnp.float32)], axis=0)
    ln2 = ln_weight.reshape(1, D)
    misc, hb = _router(x, gwc, ln2)
    c = _cumsum(misc)
    posm, bem = _dispatch(misc, c)
    idx = posm[:, :DSPLIT].reshape(1, DSPLIT * 2 * S)
    be = bem[:NB, 0]

    def _to_words(a, rows):
        return jax.lax.bitcast_convert_type(
            a.reshape(rows, DCW, 2), jnp.int32)

    def _from_words(a, rows, cols):
        return jax.lax.bitcast_convert_type(a, jnp.bfloat16).reshape(rows, cols)

    xs = _sc_scatter(_to_words(hb, DSPLIT * S), idx)
    ys = _grouped_mlp(be, _from_words(xs, NP, D),
                      expert_gate_w.astype(jnp.bfloat16),
                      expert_up_w.astype(jnp.bfloat16),
                      expert_down_w.astype(jnp.bfloat16))
    g = _from_words(_sc_gather(_to_words(ys, NP * DSPLIT), idx), 2 * S, D)
    ysh = _shared(hb,
                  shared_gate_w.astype(jnp.bfloat16),
                  shared_up_w.astype(jnp.bfloat16),
                  shared_down_w.astype(jnp.bfloat16))
    out = _final(x, ysh, g[:S], g[S:], misc)
    return out.reshape(1, S, D)


# speedup vs baseline: 24.1819x; 24.1819x over previous
import jax
import jax.numpy as jnp
from jax.experimental import pallas as pl
from jax.experimental.pallas import tpu as pltpu
from jax.experimental.pallas import tpu_sc as plsc

E = 8
D = 2048
DFF = 1408
DSH = 5632
S = 2048
EPS = 1e-06
BLK = 256
NB = 24
NP = NB * BLK
LANES = 128
NEG = -1e30
RT = 512
TT = 512
HT = 1408
SCG = 128
NCH = 4
CW = 256


def _pack_chunks(y, out_ref):
    b = jax.lax.bitcast_convert_type(y, jnp.int32)

    def rne(v):
        return jax.lax.shift_right_logical(
            v + 0x7FFF + (jax.lax.shift_right_logical(v, 16) & 1), 16)

    for j in range(NCH):
        lo = rne(b[:, 2 * CW * j:2 * CW * j + CW])
        hi = rne(b[:, 2 * CW * j + CW:2 * CW * (j + 1)])
        out_ref[j] = lo | jax.lax.shift_left(hi, 16)


def _unpack_chunk(w):
    lo = jax.lax.bitcast_convert_type(jax.lax.shift_left(w, 16), jnp.float32)
    hi = jax.lax.bitcast_convert_type(w & jnp.int32(-65536), jnp.float32)
    return jnp.concatenate([lo, hi], axis=1)


def _unpack_rows(w_ref):
    return jnp.concatenate([_unpack_chunk(w_ref[j]) for j in range(NCH)],
                           axis=1)


def _router_body(x_ref, gwc_ref, ln_ref, misc_ref, hb_ref, hw_ref):
    x = x_ref[...]
    var = jnp.mean(x * x, axis=1, keepdims=True)
    h = x * jax.lax.rsqrt(var + EPS) * ln_ref[...]
    hb_ref[...] = h.astype(jnp.bfloat16)
    _pack_chunks(h, hw_ref)
    logits = jax.lax.dot_general(
        h.astype(jnp.bfloat16), gwc_ref[...].astype(jnp.bfloat16),
        (((1,), (1,)), ((), ())),
        preferred_element_type=jnp.float32)
    lane = jax.lax.broadcasted_iota(jnp.int32, logits.shape, 1)
    l1 = jnp.where(lane < E, logits, NEG)
    m1 = jnp.max(l1, axis=1, keepdims=True)
    a1 = jnp.min(jnp.where(l1 == m1, lane, LANES), axis=1, keepdims=True)
    l2 = jnp.where(lane == a1, NEG, l1)
    m2 = jnp.max(l2, axis=1, keepdims=True)
    a2 = jnp.min(jnp.where(l2 == m2, lane, LANES), axis=1, keepdims=True)
    s2 = jnp.exp(m2 - m1)
    w0 = 1.0 / (1.0 + s2)
    w1 = 1.0 - w0
    sig = jax.nn.sigmoid(logits[:, E:E + 1])
    misc_ref[...] = (jnp.where(lane == 0, w0, 0.0)
                     + jnp.where(lane == 1, w1, 0.0)
                     + jnp.where(lane == 2, a1.astype(jnp.float32), 0.0)
                     + jnp.where(lane == 3, a2.astype(jnp.float32), 0.0)
                     + jnp.where(lane == 4, sig, 0.0))


def _router(x, gwc, ln2):
    return pl.pallas_call(
        _router_body,
        grid=(S // TT,),
        in_specs=[
            pl.BlockSpec((TT, D), lambda i: (i, 0)),
            pl.BlockSpec((LANES, D), lambda i: (0, 0)),
            pl.BlockSpec((1, D), lambda i: (0, 0)),
        ],
        out_specs=[
            pl.BlockSpec((TT, LANES), lambda i: (i, 0)),
            pl.BlockSpec((TT, D), lambda i: (i, 0)),
            pl.BlockSpec((NCH, TT, CW), lambda i: (0, i, 0)),
        ],
        out_shape=[
            jax.ShapeDtypeStruct((S, LANES), jnp.float32),
            jax.ShapeDtypeStruct((S, D), jnp.bfloat16),
            jax.ShapeDtypeStruct((NCH, S, CW), jnp.int32),
        ],
    )(x, gwc, ln2)


def _onehots(m):
    lane = jax.lax.broadcasted_iota(jnp.int32, (S, LANES), 1)
    e0 = m[:, 2:3].astype(jnp.int32)
    e1 = m[:, 3:4].astype(jnp.int32)
    return lane == e0, lane == e1


def _cumsum_body(misc_ref, c_ref):
    i = pl.program_id(0)
    o0, o1 = _onehots(misc_ref[...])
    onehot = jnp.concatenate(
        [o0.astype(jnp.bfloat16), o1.astype(jnp.bfloat16)], axis=0)
    row = jax.lax.broadcasted_iota(jnp.int32, (RT, 2 * S), 0) + i * RT
    col = jax.lax.broadcasted_iota(jnp.int32, (RT, 2 * S), 1)
    tri = (col <= row).astype(jnp.bfloat16)
    c_ref[...] = jax.lax.dot_general(
        tri, onehot, (((1,), (0,)), ((), ())),
        preferred_element_type=jnp.float32)


def _cumsum(misc):
    return pl.pallas_call(
        _cumsum_body,
        grid=(2 * S // RT,),
        in_specs=[pl.BlockSpec((S, LANES), lambda i: (0, 0))],
        out_specs=pl.BlockSpec((RT, LANES), lambda i: (i, 0)),
        out_shape=jax.ShapeDtypeStruct((2 * S, LANES), jnp.float32),
    )(misc)


def _dispatch_body(misc_ref, c_ref, pos_ref, be_ref):
    m = misc_ref[...]
    c = c_ref[...]
    o0, o1 = _onehots(m)
    counts = c[2 * S - 1:2 * S, :]
    nb = (counts.astype(jnp.int32) + BLK - 1) // BLK
    el = jax.lax.broadcasted_iota(jnp.int32, (LANES, LANES), 0)
    ec = jax.lax.broadcasted_iota(jnp.int32, (LANES, LANES), 1)
    tri = (el < ec).astype(jnp.float32)
    cumnb = jax.lax.dot_general(
        nb.astype(jnp.float32), tri, (((1,), (0,)), ((), ())),
        preferred_element_type=jnp.float32,
        precision=jax.lax.Precision.HIGHEST)
    base = BLK * cumnb
    p0 = jnp.sum(jnp.where(o0, c[0:S, :] - 1.0 + base, 0.0),
                 axis=1, keepdims=True)
    p1 = jnp.sum(jnp.where(o1, c[S:2 * S, :] - 1.0 + base, 0.0),
                 axis=1, keepdims=True)
    posm = jnp.concatenate([p0, p1], axis=0)
    lane2 = jax.lax.broadcasted_iota(jnp.int32, (2 * S, LANES), 1)
    lane_f = lane2.astype(jnp.float32)
    pos_ref[...] = jnp.where(lane2 < NCH, NP * lane_f + posm,
                             0.0).astype(jnp.int32)
    brow = jax.lax.broadcasted_iota(jnp.int32, (32, LANES), 0)
    lane_b = jax.lax.broadcasted_iota(jnp.int32, (32, LANES), 1)
    active = jnp.where((lane_b < E) & (cumnb <= brow.astype(jnp.float32)),
                       1, 0)
    be = jnp.sum(active, axis=1, keepdims=True) - 1
    total = jnp.sum(jnp.where(lane_b == E, cumnb, 0.0),
                    axis=1, keepdims=True).astype(jnp.int32)
    be_ref[...] = jnp.where(lane_b == 0, be,
                            jnp.where(lane_b == 1, total, 0))


def _dispatch(misc, c):
    return pl.pallas_call(
        _dispatch_body,
        out_shape=[
            jax.ShapeDtypeStruct((2 * S, LANES), jnp.int32),
            jax.ShapeDtypeStruct((32, LANES), jnp.int32),
        ],
    )(misc, c)


def _sc_mesh():
    return plsc.VectorSubcoreMesh(core_axis_name="core",
                                  subcore_axis_name="subcore")


N_IDX = 2 * NCH * S
N_SRC = NCH * S // SCG


def _sc_scatter(hw2, idx):
    @pl.kernel(out_type=jax.ShapeDtypeStruct((NCH * NP, CW), jnp.int32),
               mesh=_sc_mesh())
    def k(x_hbm, i_hbm, o_hbm):
        def body(x_vmem, i_vmem):
            pltpu.sync_copy(x_vmem, o_hbm.at[i_vmem.at[0]])

        pltpu.emit_pipeline(
            body,
            grid=(N_IDX // SCG,),
            in_specs=[
                pl.BlockSpec((SCG, CW), lambda i: (i % N_SRC, 0)),
                pl.BlockSpec((1, SCG), lambda i: (0, i)),
            ],
            out_specs=[],
            core_axis_name=("core", "subcore"),
            dimension_semantics=(pltpu.PARALLEL,),
        )(x_hbm, i_hbm)

    return k(hw2, idx)


def _sc_gather(ys2, idx):
    @pl.kernel(out_type=jax.ShapeDtypeStruct((N_IDX, CW), jnp.int32),
               mesh=_sc_mesh())
    def k(y_hbm, i_hbm, o_hbm):
        def body(i_vmem, o_vmem):
            pltpu.sync_copy(y_hbm.at[i_vmem.at[0]], o_vmem)

        pltpu.emit_pipeline(
            body,
            grid=(N_IDX // SCG,),
            in_specs=[pl.BlockSpec((1, SCG), lambda i: (0, i))],
            out_specs=[pl.BlockSpec((SCG, CW), lambda i: (i, 0))],
            core_axis_name=("core", "subcore"),
            dimension_semantics=(pltpu.PARALLEL,),
        )(i_hbm, o_hbm)

    return k(ys2, idx)


def _grouped_body(be_ref, x_ref, gw_ref, uw_ref, dw_ref, y_ref):
    @pl.when(pl.program_id(0) < be_ref[0, 1])
    def _():
        x = _unpack_rows(x_ref).astype(jnp.bfloat16)
        g = jax.lax.dot_general(x, gw_ref[0], (((1,), (1,)), ((), ())),
                                preferred_element_type=jnp.float32)
        u = jax.lax.dot_general(x, uw_ref[0], (((1,), (1,)), ((), ())),
                                preferred_element_type=jnp.float32)
        p = (jax.nn.silu(g) * u).astype(jnp.bfloat16)
        y = jax.lax.dot_general(p, dw_ref[0], (((1,), (1,)), ((), ())),
                                preferred_element_type=jnp.float32)
        _pack_chunks(y, y_ref)


def _grouped_mlp(be, xs, egw, euw, edw):
    grid_spec = pltpu.PrefetchScalarGridSpec(
        num_scalar_prefetch=1,
        grid=(NB,),
        in_specs=[
            pl.BlockSpec((NCH, BLK, CW), lambda b, be_ref: (0, b, 0)),
            pl.BlockSpec((1, DFF, D), lambda b, be_ref: (be_ref[b, 0], 0, 0)),
            pl.BlockSpec((1, DFF, D), lambda b, be_ref: (be_ref[b, 0], 0, 0)),
            pl.BlockSpec((1, D, DFF), lambda b, be_ref: (be_ref[b, 0], 0, 0)),
        ],
        out_specs=pl.BlockSpec((NCH, BLK, CW), lambda b, be_ref: (0, b, 0)),
    )
    return pl.pallas_call(
        _grouped_body,
        grid_spec=grid_spec,
        out_shape=jax.ShapeDtypeStruct((NCH, NP, CW), jnp.int32),
    )(be, xs, egw, euw, edw)


def _shared_body(hb_ref, sg_ref, su_ref, sd_ref, o_ref):
    @pl.when(pl.program_id(1) == 0)
    def _():
        o_ref[...] = jnp.zeros_like(o_ref)

    hb = hb_ref[...]
    g = jax.lax.dot_general(hb, sg_ref[...], (((1,), (1,)), ((), ())),
                            preferred_element_type=jnp.float32)
    u = jax.lax.dot_general(hb, su_ref[...], (((1,), (1,)), ((), ())),
                            preferred_element_type=jnp.float32)
    p = (jax.nn.silu(g) * u).astype(jnp.bfloat16)
    o_ref[...] += jax.lax.dot_general(p, sd_ref[...], (((1,), (1,)), ((), ())),
                                      preferred_element_type=jnp.float32)


def _shared(hb, sgw, suw, sdw):
    return pl.pallas_call(
        _shared_body,
        grid=(S // TT, DSH // HT),
        in_specs=[
            pl.BlockSpec((TT, D), lambda i, j: (i, 0)),
            pl.BlockSpec((HT, D), lambda i, j: (j, 0)),
            pl.BlockSpec((HT, D), lambda i, j: (j, 0)),
            pl.BlockSpec((D, HT), lambda i, j: (0, j)),
        ],
        out_specs=pl.BlockSpec((TT, D), lambda i, j: (i, 0)),
        out_shape=jax.ShapeDtypeStruct((S, D), jnp.float32),
    )(hb, sgw, suw, sdw)


def _final_body(x_ref, ysh_ref, g_ref, misc_ref, o_ref):
    m = misc_ref[...]
    w0 = m[:, 0:1]
    w1 = m[:, 1:2]
    sig = m[:, 4:5]
    g0 = jnp.concatenate([_unpack_chunk(g_ref[0, j]) for j in range(NCH)],
                         axis=1)
    g1 = jnp.concatenate([_unpack_chunk(g_ref[1, j]) for j in range(NCH)],
                         axis=1)
    o_ref[...] = (x_ref[...] + sig * ysh_ref[...] + w0 * g0 + w1 * g1)


def _final(x, ysh, g, misc):
    return pl.pallas_call(
        _final_body,
        grid=(S // TT,),
        in_specs=[
            pl.BlockSpec((TT, D), lambda i: (i, 0)),
            pl.BlockSpec((TT, D), lambda i: (i, 0)),
            pl.BlockSpec((2, NCH, TT, CW), lambda i: (0, 0, i, 0)),
            pl.BlockSpec((TT, LANES), lambda i: (i, 0)),
        ],
        out_specs=pl.BlockSpec((TT, D), lambda i: (i, 0)),
        out_shape=jax.ShapeDtypeStruct((S, D), jnp.float32),
    )(x, ysh, g, misc)


def kernel(hidden_states, ln_weight, gate_weight, expert_gate_w, expert_up_w,
           expert_down_w, shared_gate_w, shared_up_w, shared_down_w,
           shared_expert_gate_w):
    x = hidden_states.reshape(S, D)
    gwc = jnp.concatenate(
        [gate_weight, shared_expert_gate_w,
         jnp.zeros((LANES - E - 1, D), jnp.float32)], axis=0)
    ln2 = ln_weight.reshape(1, D)
    misc, hb, hw = _router(x, gwc, ln2)
    c = _cumsum(misc)
    posm, bem = _dispatch(misc, c)
    idx = (posm[:, :NCH].reshape(2, S, NCH).transpose(0, 2, 1)
           .reshape(1, N_IDX))
    be = bem[:NB, :2]
    xs = _sc_scatter(hw.reshape(NCH * S, CW), idx)
    ys = _grouped_mlp(be, xs.reshape(NCH, NP, CW),
                      expert_gate_w.astype(jnp.bfloat16),
                      expert_up_w.astype(jnp.bfloat16),
                      expert_down_w.astype(jnp.bfloat16))
    g = _sc_gather(ys.reshape(NCH * NP, CW), idx)
    ysh = _shared(hb,
                  shared_gate_w.astype(jnp.bfloat16),
                  shared_up_w.astype(jnp.bfloat16),
                  shared_down_w.astype(jnp.bfloat16))
    out = _final(x, ysh, g.reshape(2, NCH, S, CW), misc)
    return out.reshape(1, S, D)


# speedup vs baseline: 26.5252x; 1.0969x over previous
import jax
import jax.numpy as jnp
from jax.experimental import pallas as pl
from jax.experimental.pallas import tpu as pltpu
from jax.experimental.pallas import tpu_sc as plsc

E = 8
D = 2048
DFF = 1408
DSH = 5632
S = 2048
EPS = 1e-06
BLK = 256
NB = 24
NP = NB * BLK
LANES = 128
NEG = -1e30
RT = 512
TT = 512
HT = 512
STT = 1024
SCG = 128
NCH = 4
CW = 256


def _pack_chunks(y, out_ref):
    b = jax.lax.bitcast_convert_type(y, jnp.int32)

    def rne(v):
        return jax.lax.shift_right_logical(
            v + 0x7FFF + (jax.lax.shift_right_logical(v, 16) & 1), 16)

    for j in range(NCH):
        lo = rne(b[:, 2 * CW * j:2 * CW * j + CW])
        hi = rne(b[:, 2 * CW * j + CW:2 * CW * (j + 1)])
        out_ref[j] = lo | jax.lax.shift_left(hi, 16)


def _unpack_chunk(w):
    lo = jax.lax.bitcast_convert_type(jax.lax.shift_left(w, 16), jnp.float32)
    hi = jax.lax.bitcast_convert_type(w & jnp.int32(-65536), jnp.float32)
    return jnp.concatenate([lo, hi], axis=1)


def _unpack_rows(w_ref):
    return jnp.concatenate([_unpack_chunk(w_ref[j]) for j in range(NCH)],
                           axis=1)


def _router_body(x_ref, gwc_ref, ln_ref, misc_ref, hb_ref, hw_ref):
    x = x_ref[...]
    var = jnp.mean(x * x, axis=1, keepdims=True)
    h = x * jax.lax.rsqrt(var + EPS) * ln_ref[...]
    hb_ref[...] = h.astype(jnp.bfloat16)
    _pack_chunks(h, hw_ref)
    logits = jax.lax.dot_general(
        h.astype(jnp.bfloat16), gwc_ref[...].astype(jnp.bfloat16),
        (((1,), (1,)), ((), ())),
        preferred_element_type=jnp.float32)
    lane = jax.lax.broadcasted_iota(jnp.int32, logits.shape, 1)
    l1 = jnp.where(lane < E, logits, NEG)
    m1 = jnp.max(l1, axis=1, keepdims=True)
    a1 = jnp.min(jnp.where(l1 == m1, lane, LANES), axis=1, keepdims=True)
    l2 = jnp.where(lane == a1, NEG, l1)
    m2 = jnp.max(l2, axis=1, keepdims=True)
    a2 = jnp.min(jnp.where(l2 == m2, lane, LANES), axis=1, keepdims=True)
    s2 = jnp.exp(m2 - m1)
    w0 = 1.0 / (1.0 + s2)
    w1 = 1.0 - w0
    sig = jax.nn.sigmoid(logits[:, E:E + 1])
    misc_ref[...] = (jnp.where(lane == 0, w0, 0.0)
                     + jnp.where(lane == 1, w1, 0.0)
                     + jnp.where(lane == 2, a1.astype(jnp.float32), 0.0)
                     + jnp.where(lane == 3, a2.astype(jnp.float32), 0.0)
                     + jnp.where(lane == 4, sig, 0.0))


def _router(x, gwc, ln2):
    return pl.pallas_call(
        _router_body,
        grid=(S // TT,),
        in_specs=[
            pl.BlockSpec((TT, D), lambda i: (i, 0)),
            pl.BlockSpec((LANES, D), lambda i: (0, 0)),
            pl.BlockSpec((1, D), lambda i: (0, 0)),
        ],
        out_specs=[
            pl.BlockSpec((TT, LANES), lambda i: (i, 0)),
            pl.BlockSpec((TT, D), lambda i: (i, 0)),
            pl.BlockSpec((NCH, TT, CW), lambda i: (0, i, 0)),
        ],
        out_shape=[
            jax.ShapeDtypeStruct((S, LANES), jnp.float32),
            jax.ShapeDtypeStruct((S, D), jnp.bfloat16),
            jax.ShapeDtypeStruct((NCH, S, CW), jnp.int32),
        ],
    )(x, gwc, ln2)


def _onehots(m):
    lane = jax.lax.broadcasted_iota(jnp.int32, (S, LANES), 1)
    e0 = m[:, 2:3].astype(jnp.int32)
    e1 = m[:, 3:4].astype(jnp.int32)
    return lane == e0, lane == e1


def _cumsum_body(misc_ref, c_ref):
    i = pl.program_id(0)
    o0, o1 = _onehots(misc_ref[...])
    onehot = jnp.concatenate(
        [o0.astype(jnp.bfloat16), o1.astype(jnp.bfloat16)], axis=0)
    row = jax.lax.broadcasted_iota(jnp.int32, (RT, 2 * S), 0) + i * RT
    col = jax.lax.broadcasted_iota(jnp.int32, (RT, 2 * S), 1)
    tri = (col <= row).astype(jnp.bfloat16)
    c_ref[...] = jax.lax.dot_general(
        tri, onehot, (((1,), (0,)), ((), ())),
        preferred_element_type=jnp.float32)


def _cumsum(misc):
    return pl.pallas_call(
        _cumsum_body,
        grid=(2 * S // RT,),
        in_specs=[pl.BlockSpec((S, LANES), lambda i: (0, 0))],
        out_specs=pl.BlockSpec((RT, LANES), lambda i: (i, 0)),
        out_shape=jax.ShapeDtypeStruct((2 * S, LANES), jnp.float32),
    )(misc)


def _dispatch_body(misc_ref, c_ref, pos_ref, be_ref):
    m = misc_ref[...]
    c = c_ref[...]
    o0, o1 = _onehots(m)
    counts = c[2 * S - 1:2 * S, :]
    nb = (counts.astype(jnp.int32) + BLK - 1) // BLK
    el = jax.lax.broadcasted_iota(jnp.int32, (LANES, LANES), 0)
    ec = jax.lax.broadcasted_iota(jnp.int32, (LANES, LANES), 1)
    tri = (el < ec).astype(jnp.float32)
    cumnb = jax.lax.dot_general(
        nb.astype(jnp.float32), tri, (((1,), (0,)), ((), ())),
        preferred_element_type=jnp.float32,
        precision=jax.lax.Precision.HIGHEST)
    base = BLK * cumnb
    p0 = jnp.sum(jnp.where(o0, c[0:S, :] - 1.0 + base, 0.0),
                 axis=1, keepdims=True)
    p1 = jnp.sum(jnp.where(o1, c[S:2 * S, :] - 1.0 + base, 0.0),
                 axis=1, keepdims=True)
    posm = jnp.concatenate([p0, p1], axis=0)
    lane2 = jax.lax.broadcasted_iota(jnp.int32, (2 * S, LANES), 1)
    lane_f = lane2.astype(jnp.float32)
    pos_ref[...] = jnp.where(lane2 < NCH, NP * lane_f + posm,
                             0.0).astype(jnp.int32)
    brow = jax.lax.broadcasted_iota(jnp.int32, (32, LANES), 0)
    lane_b = jax.lax.broadcasted_iota(jnp.int32, (32, LANES), 1)
    active = jnp.where((lane_b < E) & (cumnb <= brow.astype(jnp.float32)),
                       1, 0)
    be = jnp.sum(active, axis=1, keepdims=True) - 1
    total = jnp.sum(jnp.where(lane_b == E, cumnb, 0.0),
                    axis=1, keepdims=True).astype(jnp.int32)
    be_ref[...] = jnp.where(lane_b == 0, be,
                            jnp.where(lane_b == 1, total, 0))


def _dispatch(misc, c):
    return pl.pallas_call(
        _dispatch_body,
        out_shape=[
            jax.ShapeDtypeStruct((2 * S, LANES), jnp.int32),
            jax.ShapeDtypeStruct((32, LANES), jnp.int32),
        ],
    )(misc, c)


def _sc_mesh():
    return plsc.VectorSubcoreMesh(core_axis_name="core",
                                  subcore_axis_name="subcore")


N_IDX = 2 * NCH * S
N_SRC = NCH * S // SCG


def _sc_scatter(hw2, idx):
    @pl.kernel(out_type=jax.ShapeDtypeStruct((NCH * NP, CW), jnp.int32),
               mesh=_sc_mesh())
    def k(x_hbm, i_hbm, o_hbm):
        def body(x_vmem, i_vmem):
            pltpu.sync_copy(x_vmem, o_hbm.at[i_vmem.at[0]])

        pltpu.emit_pipeline(
            body,
            grid=(N_IDX // SCG,),
            in_specs=[
                pl.BlockSpec((SCG, CW), lambda i: (i % N_SRC, 0)),
                pl.BlockSpec((1, SCG), lambda i: (0, i)),
            ],
            out_specs=[],
            core_axis_name=("core", "subcore"),
            dimension_semantics=(pltpu.PARALLEL,),
        )(x_hbm, i_hbm)

    return k(hw2, idx)


def _sc_gather(ys2, idx):
    @pl.kernel(out_type=jax.ShapeDtypeStruct((N_IDX, CW), jnp.int32),
               mesh=_sc_mesh())
    def k(y_hbm, i_hbm, o_hbm):
        def body(i_vmem, o_vmem):
            pltpu.sync_copy(y_hbm.at[i_vmem.at[0]], o_vmem)

        pltpu.emit_pipeline(
            body,
            grid=(N_IDX // SCG,),
            in_specs=[pl.BlockSpec((1, SCG), lambda i: (0, i))],
            out_specs=[pl.BlockSpec((SCG, CW), lambda i: (i, 0))],
            core_axis_name=("core", "subcore"),
            dimension_semantics=(pltpu.PARALLEL,),
        )(i_hbm, o_hbm)

    return k(ys2, idx)


DC2 = 2 * CW


def _grouped_body(be_ref, x_ref, gw_ref, uw_ref, dw_ref, y_ref,
                  g_scr, u_scr, p_scr):
    s = pl.program_id(1)

    @pl.when(pl.program_id(0) < be_ref[0, 1])
    def _():
        @pl.when(s < NCH)
        def _():
            xd = _unpack_chunk(x_ref[s]).astype(jnp.bfloat16)
            gp = jax.lax.dot_general(xd, gw_ref[0].astype(jnp.bfloat16),
                                     (((1,), (1,)), ((), ())),
                                     preferred_element_type=jnp.float32)
            up = jax.lax.dot_general(xd, uw_ref[0].astype(jnp.bfloat16),
                                     (((1,), (1,)), ((), ())),
                                     preferred_element_type=jnp.float32)

            @pl.when(s == 0)
            def _():
                g_scr[...] = gp
                u_scr[...] = up

            @pl.when((s > 0) & (s < NCH - 1))
            def _():
                g_scr[...] += gp
                u_scr[...] += up

            @pl.when(s == NCH - 1)
            def _():
                g = g_scr[...] + gp
                u = u_scr[...] + up
                p_scr[...] = (jax.nn.silu(g) * u).astype(jnp.bfloat16)

        @pl.when(s == NCH)
        def _():
            y = jax.lax.dot_general(p_scr[...], dw_ref[0].astype(jnp.bfloat16),
                                    (((1,), (1,)), ((), ())),
                                    preferred_element_type=jnp.float32)
            _pack_chunks(y, y_ref)


def _grouped_mlp(be, xs, egw, euw, edw):
    def _wq(b, s, be_ref):
        return (be_ref[b, 0], 0, jnp.minimum(s, NCH - 1))

    grid_spec = pltpu.PrefetchScalarGridSpec(
        num_scalar_prefetch=1,
        grid=(NB, NCH + 1),
        in_specs=[
            pl.BlockSpec((NCH, BLK, CW), lambda b, s, be_ref: (0, b, 0)),
            pl.BlockSpec((1, DFF, DC2), _wq),
            pl.BlockSpec((1, DFF, DC2), _wq),
            pl.BlockSpec((1, D, DFF), lambda b, s, be_ref: (be_ref[b, 0], 0, 0)),
        ],
        out_specs=pl.BlockSpec((NCH, BLK, CW), lambda b, s, be_ref: (0, b, 0)),
        scratch_shapes=[
            pltpu.VMEM((BLK, DFF), jnp.float32),
            pltpu.VMEM((BLK, DFF), jnp.float32),
            pltpu.VMEM((BLK, DFF), jnp.bfloat16),
        ],
    )
    return pl.pallas_call(
        _grouped_body,
        grid_spec=grid_spec,
        out_shape=jax.ShapeDtypeStruct((NCH, NP, CW), jnp.int32),
    )(be, xs, egw, euw, edw)


def _shared_body(hb_ref, sg_ref, su_ref, sd_ref, o_ref):
    @pl.when(pl.program_id(1) == 0)
    def _():
        o_ref[...] = jnp.zeros_like(o_ref)

    hb = hb_ref[...]
    g = jax.lax.dot_general(hb, sg_ref[...].astype(jnp.bfloat16),
                            (((1,), (1,)), ((), ())),
                            preferred_element_type=jnp.float32)
    u = jax.lax.dot_general(hb, su_ref[...].astype(jnp.bfloat16),
                            (((1,), (1,)), ((), ())),
                            preferred_element_type=jnp.float32)
    p = (jax.nn.silu(g) * u).astype(jnp.bfloat16)
    o_ref[...] += jax.lax.dot_general(p, sd_ref[...].astype(jnp.bfloat16),
                                      (((1,), (1,)), ((), ())),
                                      preferred_element_type=jnp.float32)


def _shared(hb, sgw, suw, sdw):
    return pl.pallas_call(
        _shared_body,
        grid=(S // STT, DSH // HT),
        in_specs=[
            pl.BlockSpec((STT, D), lambda i, j: (i, 0)),
            pl.BlockSpec((HT, D), lambda i, j: (j, 0)),
            pl.BlockSpec((HT, D), lambda i, j: (j, 0)),
            pl.BlockSpec((D, HT), lambda i, j: (0, j)),
        ],
        out_specs=pl.BlockSpec((STT, D), lambda i, j: (i, 0)),
        out_shape=jax.ShapeDtypeStruct((S, D), jnp.float32),
    )(hb, sgw, suw, sdw)


def _final_body(x_ref, ysh_ref, g_ref, misc_ref, o_ref):
    m = misc_ref[...]
    w0 = m[:, 0:1]
    w1 = m[:, 1:2]
    sig = m[:, 4:5]
    g0 = jnp.concatenate([_unpack_chunk(g_ref[0, j]) for j in range(NCH)],
                         axis=1)
    g1 = jnp.concatenate([_unpack_chunk(g_ref[1, j]) for j in range(NCH)],
                         axis=1)
    o_ref[...] = (x_ref[...] + sig * ysh_ref[...] + w0 * g0 + w1 * g1)


def _final(x, ysh, g, misc):
    return pl.pallas_call(
        _final_body,
        grid=(S // TT,),
        in_specs=[
            pl.BlockSpec((TT, D), lambda i: (i, 0)),
            pl.BlockSpec((TT, D), lambda i: (i, 0)),
            pl.BlockSpec((2, NCH, TT, CW), lambda i: (0, 0, i, 0)),
            pl.BlockSpec((TT, LANES), lambda i: (i, 0)),
        ],
        out_specs=pl.BlockSpec((TT, D), lambda i: (i, 0)),
        out_shape=jax.ShapeDtypeStruct((S, D), jnp.float32),
    )(x, ysh, g, misc)


def kernel(hidden_states, ln_weight, gate_weight, expert_gate_w, expert_up_w,
           expert_down_w, shared_gate_w, shared_up_w, shared_down_w,
           shared_expert_gate_w):
    x = hidden_states.reshape(S, D)
    gwc = jnp.concatenate(
        [gate_weight, shared_expert_gate_w,
         jnp.zeros((LANES - E - 1, D), jnp.float32)], axis=0)
    ln2 = ln_weight.reshape(1, D)
    misc, hb, hw = _router(x, gwc, ln2)
    c = _cumsum(misc)
    posm, bem = _dispatch(misc, c)
    idx = (posm[:, :NCH].reshape(2, S, NCH).transpose(0, 2, 1)
           .reshape(1, N_IDX))
    be = bem[:NB, :2]
    xs = _sc_scatter(hw.reshape(NCH * S, CW), idx)
    ys = _grouped_mlp(be, xs.reshape(NCH, NP, CW),
                      expert_gate_w, expert_up_w, expert_down_w)
    g = _sc_gather(ys.reshape(NCH * NP, CW), idx)
    ysh = _shared(hb, shared_gate_w, shared_up_w, shared_down_w)
    out = _final(x, ysh, g.reshape(2, NCH, S, CW), misc)
    return out.reshape(1, S, D)
